# baseline (speedup 1.0000x reference)
import functools

import jax
import jax.numpy as jnp
from jax import lax
from jax.experimental import pallas as pl
from jax.experimental.pallas import tpu as pltpu
from jax.experimental.pallas import tpu_sc as plsc

N = 10000
E = 160000
D = 256
HALF = 128
EPS = 1e-5

NSUB = 16
EPSUB = E // NSUB
CH = 80
NCHUNK = EPSUB // CH
NBUF = 4
IBUF = 8
ILEAD = 6
PERIOD = 8
NLOOP = (NCHUNK // PERIOD) * PERIOD
ZR = 48
RPS = 624
TAIL = N - NSUB * RPS

RB = 1000
GRID = N // RB


def _sc_agg(x2, ssl, dsl, ssr, dsr):
    mesh = plsc.VectorSubcoreMesh(core_axis_name="c", subcore_axis_name="s")
    half = jax.ShapeDtypeStruct((N, HALF), jnp.float32)

    @functools.partial(
        pl.kernel,
        out_type=(half, half, half, half),
        mesh=mesh,
        scratch_types=(
            (pltpu.VMEM((CH,), jnp.int32),) * IBUF,
            (pltpu.VMEM((CH,), jnp.int32),) * IBUF,
            (pltpu.VMEM((CH, HALF), jnp.float32),) * NBUF,
            pltpu.VMEM((ZR, HALF), jnp.float32),
            pltpu.VMEM_SHARED((N, HALF), jnp.float32),
            (pltpu.SemaphoreType.DMA,) * NBUF,
            (pltpu.SemaphoreType.DMA,) * IBUF,
            (pltpu.SemaphoreType.DMA,) * NBUF,
            pltpu.SemaphoreType.DMA,
        ),
    )
    def k(x2_h, ssl_h, dsl_h, ssr_h, dsr_h,
          osl_lo, osl_hi, osr_lo, osr_hi,
          src_v, dst_v, rows_v, zero_v, acc, gsems, isems, ssems, semz):
        s = lax.axis_index("s")
        c = lax.axis_index("c")

        z16 = jnp.zeros((16,), jnp.float32)
        for r in range(ZR):
            for q in range(HALF // 16):
                zero_v[r, pl.ds(q * 16, 16)] = z16

        def run_half(o_sl, o_sr):
            r0 = s * RPS
            last = s == NSUB - 1

            def fire_zero_own():
                for z in range(RPS // ZR):
                    pltpu.async_copy(zero_v, acc.at[pl.ds(r0 + z * ZR, ZR)],
                                     semz)

                @pl.when(last)
                def _():
                    pltpu.async_copy(zero_v.at[pl.ds(0, TAIL)],
                                     acc.at[pl.ds(NSUB * RPS, TAIL)], semz)

            def drain_zero_own():
                for z in range(RPS // ZR):
                    pltpu.make_async_copy(zero_v, acc.at[pl.ds(r0, ZR)],
                                          semz).wait()

                @pl.when(last)
                def _():
                    pltpu.make_async_copy(zero_v.at[pl.ds(0, TAIL)],
                                          acc.at[pl.ds(NSUB * RPS, TAIL)],
                                          semz).wait()

            for pi, (sr_h, dr_h, o) in enumerate(
                    ((ssl_h, dsl_h, o_sl), (ssr_h, dsr_h, o_sr))):
                e0 = s * EPSUB

                def start_idx(j, b):
                    pltpu.async_copy(sr_h.at[pl.ds(e0 + j * CH, CH)],
                                     src_v[b], isems[b])
                    pltpu.async_copy(dr_h.at[pl.ds(e0 + j * CH, CH)],
                                     dst_v[b], isems[b])

                def wait_idx(b):
                    pltpu.make_async_copy(sr_h.at[pl.ds(e0, CH)],
                                          src_v[b], isems[b]).wait()
                    pltpu.make_async_copy(dr_h.at[pl.ds(e0, CH)],
                                          dst_v[b], isems[b]).wait()
                    for q in range(CH // 16):
                        v = src_v[b][pl.ds(q * 16, 16)]
                        src_v[b][pl.ds(q * 16, 16)] = v + v + c

                def start_gather(b, bi):
                    pltpu.async_copy(x2_h.at[src_v[bi]], rows_v[b], gsems[b])

                def wait_gather(b):
                    pltpu.make_async_copy(x2_h.at[src_v[0]], rows_v[b],
                                          gsems[b]).wait()

                def start_scatter(b, bi):
                    pltpu.async_copy(rows_v[b], acc.at[dst_v[bi]], ssems[b],
                                     add=True)

                def wait_scatter(b):
                    pltpu.make_async_copy(rows_v[b], acc.at[dst_v[0]],
                                          ssems[b]).wait()

                if pi == 0:
                    fire_zero_own()
                    for b in range(ILEAD):
                        start_idx(b, b)
                    drain_zero_own()
                    plsc.subcore_barrier()
                else:
                    for b in range(ILEAD):
                        start_idx(b, b)

                GLEAD = 2
                for b in range(GLEAD):
                    wait_idx(b)
                    start_gather(b, b)

                @pl.loop(0, NLOOP, step=PERIOD)
                def _(g):
                    for b8 in range(PERIOD):
                        j = g + b8
                        b = b8 % NBUF
                        bi = b8 % IBUF
                        wait_gather(b)
                        start_scatter(b, bi)

                        @pl.when(j + GLEAD < NCHUNK)
                        def _():
                            bg = (b + GLEAD) % NBUF
                            big = (bi + GLEAD) % IBUF

                            @pl.when(j >= GLEAD)
                            def _():
                                wait_scatter(bg)

                            wait_idx(big)
                            start_gather(bg, big)

                        @pl.when(j + ILEAD < NCHUNK)
                        def _():
                            start_idx(j + ILEAD, (bi + ILEAD) % IBUF)

                for j in range(NLOOP, NCHUNK):
                    b = j % NBUF
                    bi = j % IBUF
                    wait_gather(b)
                    start_scatter(b, bi)
                    if j + GLEAD < NCHUNK:
                        wait_scatter((b + GLEAD) % NBUF)
                        wait_idx((bi + GLEAD) % IBUF)
                        start_gather((b + GLEAD) % NBUF, (bi + GLEAD) % IBUF)
                    if j + ILEAD < NCHUNK:
                        start_idx(j + ILEAD, (bi + ILEAD) % IBUF)

                for j in range(NCHUNK - NBUF, NCHUNK):
                    wait_scatter(j % NBUF)

                plsc.subcore_barrier()
                pltpu.sync_copy(acc.at[pl.ds(r0, RPS)], o.at[pl.ds(r0, RPS)])

                @pl.when(last)
                def _():
                    pltpu.sync_copy(acc.at[pl.ds(NSUB * RPS, TAIL)],
                                    o.at[pl.ds(NSUB * RPS, TAIL)])

                if pi == 0:
                    fire_zero_own()
                    drain_zero_own()
                    plsc.subcore_barrier()

        @pl.when(c == 0)
        def _():
            run_half(osl_lo, osr_lo)

        @pl.when(c == 1)
        def _():
            run_half(osl_hi, osr_hi)

    return k(x2, ssl, dsl, ssr, dsr)


def _dot(a, w):
    return lax.dot_general(a, w, (((1,), (1,)), ((), ())),
                           precision=lax.Precision.DEFAULT,
                           preferred_element_type=jnp.float32)


def _mm_body(x, aslo, ashi, asrlo, asrhi,
             wslt, wsrt, wrslt, wrsrt, bias, y_ref, st_ref):
    i = pl.program_id(0)
    wroot = wrslt[...] + wrsrt[...]
    wsl = wslt[...]
    wsr = wsrt[...]
    y = (_dot(aslo[...], wsl[:, :HALF]) + _dot(ashi[...], wsl[:, HALF:])
         + _dot(asrlo[...], wsr[:, :HALF]) + _dot(asrhi[...], wsr[:, HALF:])
         + _dot(x[...], wroot)
         + bias[...])
    y_ref[...] = y
    s1 = jnp.sum(y, axis=0, keepdims=True)
    s2 = jnp.sum(y * y, axis=0, keepdims=True)
    st = jnp.concatenate([s1, s2], axis=0)

    @pl.when(i == 0)
    def _():
        st_ref[...] = st

    @pl.when(i > 0)
    def _():
        st_ref[...] = st_ref[...] + st


def _tc_mm(x, aslo, ashi, asrlo, asrhi, Wsl, Wsr, Wrsl, Wrsr, bsl, bsr):
    hblk = lambda i: (i, 0)
    full = lambda i: (0, 0)
    bias = (bsl + bsr).reshape(1, D)
    return pl.pallas_call(
        _mm_body,
        grid=(GRID,),
        in_specs=[
            pl.BlockSpec((RB, D), hblk),
            pl.BlockSpec((RB, HALF), hblk),
            pl.BlockSpec((RB, HALF), hblk),
            pl.BlockSpec((RB, HALF), hblk),
            pl.BlockSpec((RB, HALF), hblk),
            pl.BlockSpec((D, D), full),
            pl.BlockSpec((D, D), full),
            pl.BlockSpec((D, D), full),
            pl.BlockSpec((D, D), full),
            pl.BlockSpec((1, D), full),
        ],
        out_specs=[
            pl.BlockSpec((RB, D), hblk),
            pl.BlockSpec((2, D), full),
        ],
        out_shape=[
            jax.ShapeDtypeStruct((N, D), jnp.float32),
            jax.ShapeDtypeStruct((2, D), jnp.float32),
        ],
    )(x, aslo, ashi, asrlo, asrhi, Wsl, Wsr, Wrsl, Wrsr, bias)


def _bn_relu(y, st, g, b):
    m = st[0:1] / N
    v = st[1:2] / N - m * m
    scale = lax.rsqrt(v + EPS) * g
    return jnp.maximum((y - m) * scale + b, 0.0)


def _bn_body(y_ref, st_ref, g_ref, b_ref, o_ref):
    o_ref[...] = _bn_relu(y_ref[...], st_ref[...], g_ref[...], b_ref[...])


def _tc_bn(y, st, g, b):
    return pl.pallas_call(
        _bn_body,
        grid=(GRID,),
        in_specs=[
            pl.BlockSpec((RB, D), lambda i: (i, 0)),
            pl.BlockSpec((2, D), lambda i: (0, 0)),
            pl.BlockSpec((1, D), lambda i: (0, 0)),
            pl.BlockSpec((1, D), lambda i: (0, 0)),
        ],
        out_specs=pl.BlockSpec((RB, D), lambda i: (i, 0)),
        out_shape=jax.ShapeDtypeStruct((N, D), jnp.float32),
    )(y, st, g.reshape(1, D), b.reshape(1, D))


def kernel(x, edge_index_sl, edge_index_sr,
           Wrel1_sl, brel1_sl, Wroot1_sl,
           Wrel1_sr, brel1_sr, Wroot1_sr,
           Wrel2_sl, brel2_sl, Wroot2_sl,
           Wrel2_sr, brel2_sr, Wroot2_sr,
           bn1_g, bn1_b, bn2_g, bn2_b):
    ssl = edge_index_sl[0]
    dsl = edge_index_sl[1]
    ssr = edge_index_sr[0]
    dsr = edge_index_sr[1]

    a1 = _sc_agg(x.reshape(2 * N, HALF), ssl, dsl, ssr, dsr)
    y1, st1 = _tc_mm(x, *a1, Wrel1_sl, Wrel1_sr,
                     Wroot1_sl, Wroot1_sr, brel1_sl, brel1_sr)
    h = _tc_bn(y1, st1, bn1_g, bn1_b)

    a2 = _sc_agg(h.reshape(2 * N, HALF), ssl, dsl, ssr, dsr)
    y2, st2 = _tc_mm(h, *a2, Wrel2_sl, Wrel2_sr,
                     Wroot2_sl, Wroot2_sr, brel2_sl, brel2_sr)
    return _tc_bn(y2, st2, bn2_g, bn2_b)


# speedup vs baseline: 1.1305x; 1.1305x over previous
import functools

import jax
import jax.numpy as jnp
from jax import lax
from jax.experimental import pallas as pl
from jax.experimental.pallas import tpu as pltpu
from jax.experimental.pallas import tpu_sc as plsc

N = 10000
E = 160000
D = 256
HALF = 128
EPS = 1e-5

NSUB = 16
EPSUB = E // NSUB
CH = 80
NCHUNK = EPSUB // CH
NBUF = 4
NLOOP = (NCHUNK // NBUF) * NBUF
ZR = 48
RPS = 624
TAIL = N - NSUB * RPS

RB = 1000
GRID = N // RB


def _sc_agg(x2, ssl, dsl, ssr, dsr):
    mesh = plsc.VectorSubcoreMesh(core_axis_name="c", subcore_axis_name="s")
    half = jax.ShapeDtypeStruct((N, HALF), jnp.float32)

    @functools.partial(
        pl.kernel,
        out_type=(half, half, half, half),
        mesh=mesh,
        scratch_types=(
            (pltpu.VMEM((CH,), jnp.int32),) * NBUF,
            (pltpu.VMEM((CH,), jnp.int32),) * NBUF,
            (pltpu.VMEM((CH, HALF), jnp.float32),) * NBUF,
            pltpu.VMEM((ZR, HALF), jnp.float32),
            pltpu.VMEM_SHARED((N, HALF), jnp.float32),
            (pltpu.SemaphoreType.DMA,) * NBUF,
            (pltpu.SemaphoreType.DMA,) * NBUF,
            pltpu.SemaphoreType.DMA,
        ),
    )
    def k(x2_h, ssl_h, dsl_h, ssr_h, dsr_h,
          osl_lo, osl_hi, osr_lo, osr_hi,
          src_v, dst_v, rows_v, zero_v, acc, gsems, isems, semz):
        s = lax.axis_index("s")
        c = lax.axis_index("c")

        z16 = jnp.zeros((16,), jnp.float32)
        for r in range(ZR):
            for q in range(HALF // 16):
                zero_v[r, pl.ds(q * 16, 16)] = z16

        def run_half(o_sl, o_sr):
            r0 = s * RPS
            last = s == NSUB - 1

            def fire_zero_own():
                for z in range(RPS // ZR):
                    pltpu.async_copy(zero_v, acc.at[pl.ds(r0 + z * ZR, ZR)],
                                     semz)

                @pl.when(last)
                def _():
                    pltpu.async_copy(zero_v.at[pl.ds(0, TAIL)],
                                     acc.at[pl.ds(NSUB * RPS, TAIL)], semz)

            def drain_zero_own():
                for z in range(RPS // ZR):
                    pltpu.make_async_copy(zero_v, acc.at[pl.ds(r0, ZR)],
                                          semz).wait()

                @pl.when(last)
                def _():
                    pltpu.make_async_copy(zero_v.at[pl.ds(0, TAIL)],
                                          acc.at[pl.ds(NSUB * RPS, TAIL)],
                                          semz).wait()

            for pi, (sr_h, dr_h, o) in enumerate(
                    ((ssl_h, dsl_h, o_sl), (ssr_h, dsr_h, o_sr))):
                e0 = s * EPSUB

                def start_idx(j, b):
                    pltpu.async_copy(sr_h.at[pl.ds(e0 + j * CH, CH)],
                                     src_v[b], isems[b])
                    pltpu.async_copy(dr_h.at[pl.ds(e0 + j * CH, CH)],
                                     dst_v[b], isems[b])

                def wait_idx(b):
                    pltpu.make_async_copy(sr_h.at[pl.ds(e0, CH)],
                                          src_v[b], isems[b]).wait()
                    pltpu.make_async_copy(dr_h.at[pl.ds(e0, CH)],
                                          dst_v[b], isems[b]).wait()
                    for q in range(CH // 16):
                        v = src_v[b][pl.ds(q * 16, 16)]
                        src_v[b][pl.ds(q * 16, 16)] = v + v + c

                def start_gather(b):
                    pltpu.async_copy(x2_h.at[src_v[b]], rows_v[b], gsems[b])

                def wait_gather(b):
                    pltpu.make_async_copy(x2_h.at[src_v[b]], rows_v[b],
                                          gsems[b]).wait()

                if pi == 0:
                    fire_zero_own()
                    for b in range(NBUF):
                        start_idx(b, b)
                    drain_zero_own()
                    plsc.subcore_barrier()
                else:
                    for b in range(NBUF):
                        start_idx(b, b)

                GLEAD = 2
                for b in range(GLEAD):
                    wait_idx(b)
                    start_gather(b)

                @pl.loop(0, NLOOP, step=NBUF)
                def _(g):
                    for b in range(NBUF):
                        j = g + b
                        wait_gather(b)

                        @pl.when(j + GLEAD < NCHUNK)
                        def _():
                            bg = (b + GLEAD) % NBUF
                            wait_idx(bg)
                            start_gather(bg)

                        pltpu.sync_copy(rows_v[b], acc.at[dst_v[b]],
                                        add=True)

                        @pl.when(j + NBUF < NCHUNK)
                        def _():
                            start_idx(j + NBUF, b)

                for j in range(NLOOP, NCHUNK):
                    b = j % NBUF
                    wait_gather(b)
                    pltpu.sync_copy(rows_v[b], acc.at[dst_v[b]], add=True)

                plsc.subcore_barrier()
                pltpu.sync_copy(acc.at[pl.ds(r0, RPS)], o.at[pl.ds(r0, RPS)])

                @pl.when(last)
                def _():
                    pltpu.sync_copy(acc.at[pl.ds(NSUB * RPS, TAIL)],
                                    o.at[pl.ds(NSUB * RPS, TAIL)])

                if pi == 0:
                    fire_zero_own()
                    drain_zero_own()
                    plsc.subcore_barrier()

        @pl.when(c == 0)
        def _():
            run_half(osl_lo, osr_lo)

        @pl.when(c == 1)
        def _():
            run_half(osl_hi, osr_hi)

    return k(x2, ssl, dsl, ssr, dsr)


def _dot(a, w):
    return lax.dot_general(a, w, (((1,), (1,)), ((), ())),
                           precision=lax.Precision.DEFAULT,
                           preferred_element_type=jnp.float32)


def _mm_body(x, aslo, ashi, asrlo, asrhi,
             wslt, wsrt, wrslt, wrsrt, bias, y_ref, st_ref):
    i = pl.program_id(0)
    wroot = wrslt[...] + wrsrt[...]
    wsl = wslt[...]
    wsr = wsrt[...]
    y = (_dot(aslo[...], wsl[:, :HALF]) + _dot(ashi[...], wsl[:, HALF:])
         + _dot(asrlo[...], wsr[:, :HALF]) + _dot(asrhi[...], wsr[:, HALF:])
         + _dot(x[...], wroot)
         + bias[...])
    y_ref[...] = y
    s1 = jnp.sum(y, axis=0, keepdims=True)
    s2 = jnp.sum(y * y, axis=0, keepdims=True)
    st = jnp.concatenate([s1, s2], axis=0)

    @pl.when(i == 0)
    def _():
        st_ref[...] = st

    @pl.when(i > 0)
    def _():
        st_ref[...] = st_ref[...] + st


def _tc_mm(x, aslo, ashi, asrlo, asrhi, Wsl, Wsr, Wrsl, Wrsr, bsl, bsr):
    hblk = lambda i: (i, 0)
    full = lambda i: (0, 0)
    bias = (bsl + bsr).reshape(1, D)
    return pl.pallas_call(
        _mm_body,
        grid=(GRID,),
        in_specs=[
            pl.BlockSpec((RB, D), hblk),
            pl.BlockSpec((RB, HALF), hblk),
            pl.BlockSpec((RB, HALF), hblk),
            pl.BlockSpec((RB, HALF), hblk),
            pl.BlockSpec((RB, HALF), hblk),
            pl.BlockSpec((D, D), full),
            pl.BlockSpec((D, D), full),
            pl.BlockSpec((D, D), full),
            pl.BlockSpec((D, D), full),
            pl.BlockSpec((1, D), full),
        ],
        out_specs=[
            pl.BlockSpec((RB, D), hblk),
            pl.BlockSpec((2, D), full),
        ],
        out_shape=[
            jax.ShapeDtypeStruct((N, D), jnp.float32),
            jax.ShapeDtypeStruct((2, D), jnp.float32),
        ],
    )(x, aslo, ashi, asrlo, asrhi, Wsl, Wsr, Wrsl, Wrsr, bias)


def _bn_relu(y, st, g, b):
    m = st[0:1] / N
    v = st[1:2] / N - m * m
    scale = lax.rsqrt(v + EPS) * g
    return jnp.maximum((y - m) * scale + b, 0.0)


def _bn_body(y_ref, st_ref, g_ref, b_ref, o_ref):
    o_ref[...] = _bn_relu(y_ref[...], st_ref[...], g_ref[...], b_ref[...])


def _tc_bn(y, st, g, b):
    return pl.pallas_call(
        _bn_body,
        grid=(GRID,),
        in_specs=[
            pl.BlockSpec((RB, D), lambda i: (i, 0)),
            pl.BlockSpec((2, D), lambda i: (0, 0)),
            pl.BlockSpec((1, D), lambda i: (0, 0)),
            pl.BlockSpec((1, D), lambda i: (0, 0)),
        ],
        out_specs=pl.BlockSpec((RB, D), lambda i: (i, 0)),
        out_shape=jax.ShapeDtypeStruct((N, D), jnp.float32),
    )(y, st, g.reshape(1, D), b.reshape(1, D))


def kernel(x, edge_index_sl, edge_index_sr,
           Wrel1_sl, brel1_sl, Wroot1_sl,
           Wrel1_sr, brel1_sr, Wroot1_sr,
           Wrel2_sl, brel2_sl, Wroot2_sl,
           Wrel2_sr, brel2_sr, Wroot2_sr,
           bn1_g, bn1_b, bn2_g, bn2_b):
    ssl = edge_index_sl[0]
    dsl = edge_index_sl[1]
    ssr = edge_index_sr[0]
    dsr = edge_index_sr[1]

    a1 = _sc_agg(x.reshape(2 * N, HALF), ssl, dsl, ssr, dsr)
    y1, st1 = _tc_mm(x, *a1, Wrel1_sl, Wrel1_sr,
                     Wroot1_sl, Wroot1_sr, brel1_sl, brel1_sr)
    h = _tc_bn(y1, st1, bn1_g, bn1_b)

    a2 = _sc_agg(h.reshape(2 * N, HALF), ssl, dsl, ssr, dsr)
    y2, st2 = _tc_mm(h, *a2, Wrel2_sl, Wrel2_sr,
                     Wroot2_sl, Wroot2_sr, brel2_sl, brel2_sr)
    return _tc_bn(y2, st2, bn2_g, bn2_b)


# speedup vs baseline: 1.1641x; 1.0296x over previous
import functools

import jax
import jax.numpy as jnp
from jax import lax
from jax.experimental import pallas as pl
from jax.experimental.pallas import tpu as pltpu
from jax.experimental.pallas import tpu_sc as plsc

N = 10000
E = 160000
D = 256
HALF = 128
EPS = 1e-5

NSUB = 16
EPSUB = E // NSUB
CH = 80
NCHUNK = EPSUB // CH
NBUF = 4
NLOOP = (NCHUNK // NBUF) * NBUF
ZR = 48
RPS = 624
TAIL = N - NSUB * RPS

RB = 1000
GRID = N // RB


def _sc_agg(x2, ssl, dsl, ssr, dsr):
    mesh = plsc.VectorSubcoreMesh(core_axis_name="c", subcore_axis_name="s")
    half = jax.ShapeDtypeStruct((N, HALF), jnp.float32)

    @functools.partial(
        pl.kernel,
        out_type=(half, half, half, half),
        mesh=mesh,
        scratch_types=(
            (pltpu.VMEM((CH,), jnp.int32),) * NBUF,
            (pltpu.VMEM((CH,), jnp.int32),) * NBUF,
            (pltpu.VMEM((CH, HALF), jnp.float32),) * NBUF,
            pltpu.VMEM((ZR, HALF), jnp.float32),
            pltpu.VMEM_SHARED((N, HALF), jnp.float32),
            (pltpu.SemaphoreType.DMA,) * NBUF,
            (pltpu.SemaphoreType.DMA,) * NBUF,
            pltpu.SemaphoreType.DMA,
        ),
    )
    def k(x2_h, ssl_h, dsl_h, ssr_h, dsr_h,
          osl_lo, osl_hi, osr_lo, osr_hi,
          src_v, dst_v, rows_v, zero_v, acc, gsems, isems, semz):
        s = lax.axis_index("s")
        c = lax.axis_index("c")

        z16 = jnp.zeros((16,), jnp.float32)
        for r in range(ZR):
            for q in range(HALF // 16):
                zero_v[r, pl.ds(q * 16, 16)] = z16

        def run_half(o_sl, o_sr):
            r0 = s * RPS
            last = s == NSUB - 1

            def fire_zero_own():
                for z in range(RPS // ZR):
                    pltpu.async_copy(zero_v, acc.at[pl.ds(r0 + z * ZR, ZR)],
                                     semz)

                @pl.when(last)
                def _():
                    pltpu.async_copy(zero_v.at[pl.ds(0, TAIL)],
                                     acc.at[pl.ds(NSUB * RPS, TAIL)], semz)

            def drain_zero_own():
                for z in range(RPS // ZR):
                    pltpu.make_async_copy(zero_v, acc.at[pl.ds(r0, ZR)],
                                          semz).wait()

                @pl.when(last)
                def _():
                    pltpu.make_async_copy(zero_v.at[pl.ds(0, TAIL)],
                                          acc.at[pl.ds(NSUB * RPS, TAIL)],
                                          semz).wait()

            for pi, (sr_h, dr_h, o) in enumerate(
                    ((ssl_h, dsl_h, o_sl), (ssr_h, dsr_h, o_sr))):
                e0 = s * EPSUB

                def start_idx(j, b):
                    pltpu.async_copy(sr_h.at[pl.ds(e0 + j * CH, CH)],
                                     src_v[b], isems[b])
                    pltpu.async_copy(dr_h.at[pl.ds(e0 + j * CH, CH)],
                                     dst_v[b], isems[b])

                def wait_idx(b):
                    pltpu.make_async_copy(sr_h.at[pl.ds(e0, CH)],
                                          src_v[b], isems[b]).wait()
                    pltpu.make_async_copy(dr_h.at[pl.ds(e0, CH)],
                                          dst_v[b], isems[b]).wait()
                    for q in range(CH // 16):
                        v = src_v[b][pl.ds(q * 16, 16)]
                        src_v[b][pl.ds(q * 16, 16)] = v + v + c

                def start_gather(b):
                    pltpu.async_copy(x2_h.at[src_v[b]], rows_v[b], gsems[b])

                def wait_gather(b):
                    pltpu.make_async_copy(x2_h.at[src_v[b]], rows_v[b],
                                          gsems[b]).wait()

                if pi == 0:
                    fire_zero_own()
                    for b in range(NBUF):
                        start_idx(b, b)
                    drain_zero_own()
                    plsc.subcore_barrier()
                else:
                    for b in range(NBUF):
                        start_idx(b, b)

                GLEAD = 2
                for b in range(GLEAD):
                    wait_idx(b)
                    start_gather(b)

                @pl.loop(0, NLOOP, step=NBUF)
                def _(g):
                    for b in range(NBUF):
                        j = g + b
                        wait_gather(b)

                        @pl.when(j + GLEAD < NCHUNK)
                        def _():
                            bg = (b + GLEAD) % NBUF
                            wait_idx(bg)
                            start_gather(bg)

                        pltpu.sync_copy(rows_v[b], acc.at[dst_v[b]],
                                        add=True)

                        @pl.when(j + NBUF < NCHUNK)
                        def _():
                            start_idx(j + NBUF, b)

                for j in range(NLOOP, NCHUNK):
                    b = j % NBUF
                    wait_gather(b)
                    pltpu.sync_copy(rows_v[b], acc.at[dst_v[b]], add=True)

                plsc.subcore_barrier()
                pltpu.sync_copy(acc.at[pl.ds(r0, RPS)], o.at[pl.ds(r0, RPS)])

                @pl.when(last)
                def _():
                    pltpu.sync_copy(acc.at[pl.ds(NSUB * RPS, TAIL)],
                                    o.at[pl.ds(NSUB * RPS, TAIL)])

                if pi == 0:
                    fire_zero_own()
                    drain_zero_own()
                    plsc.subcore_barrier()

        @pl.when(c == 0)
        def _():
            run_half(osl_lo, osr_lo)

        @pl.when(c == 1)
        def _():
            run_half(osl_hi, osr_hi)

    return k(x2, ssl, dsl, ssr, dsr)


def _dot(a, w):
    return lax.dot_general(a, w, (((1,), (1,)), ((), ())),
                           precision=lax.Precision.DEFAULT,
                           preferred_element_type=jnp.float32)


def _layer_body(x, aslo, ashi, asrlo, asrhi,
                wslt, wsrt, wrslt, wrsrt, bias, g_ref, b_ref, o_ref):
    wroot = wrslt[...] + wrsrt[...]
    wsl = wslt[...]
    wsr = wsrt[...]
    y = (_dot(aslo[...], wsl[:, :HALF]) + _dot(ashi[...], wsl[:, HALF:])
         + _dot(asrlo[...], wsr[:, :HALF]) + _dot(asrhi[...], wsr[:, HALF:])
         + _dot(x[...], wroot)
         + bias[...])
    m = jnp.mean(y, axis=0, keepdims=True)
    v = jnp.mean(y * y, axis=0, keepdims=True) - m * m
    scale = lax.rsqrt(v + EPS) * g_ref[...]
    r = jnp.maximum((y - m) * scale + b_ref[...], 0.0)
    o_ref[...] = r.astype(o_ref.dtype)


def _tc_layer(x, aslo, ashi, asrlo, asrhi, Wsl, Wsr, Wrsl, Wrsr,
              bsl, bsr, g, b, dtype):
    bias = (bsl + bsr).reshape(1, D)
    return pl.pallas_call(
        _layer_body,
        out_shape=jax.ShapeDtypeStruct((N, D), dtype),
    )(x, aslo, ashi, asrlo, asrhi, Wsl, Wsr, Wrsl, Wrsr, bias,
      g.reshape(1, D), b.reshape(1, D))


def _bn_relu(y, st, g, b):
    m = st[0:1] / N
    v = st[1:2] / N - m * m
    scale = lax.rsqrt(v + EPS) * g
    return jnp.maximum((y - m) * scale + b, 0.0)


def _bn_body(y_ref, st_ref, g_ref, b_ref, o_ref):
    o_ref[...] = _bn_relu(y_ref[...], st_ref[...], g_ref[...], b_ref[...])


def _tc_bn(y, st, g, b):
    return pl.pallas_call(
        _bn_body,
        grid=(GRID,),
        in_specs=[
            pl.BlockSpec((RB, D), lambda i: (i, 0)),
            pl.BlockSpec((2, D), lambda i: (0, 0)),
            pl.BlockSpec((1, D), lambda i: (0, 0)),
            pl.BlockSpec((1, D), lambda i: (0, 0)),
        ],
        out_specs=pl.BlockSpec((RB, D), lambda i: (i, 0)),
        out_shape=jax.ShapeDtypeStruct((N, D), jnp.float32),
    )(y, st, g.reshape(1, D), b.reshape(1, D))


def kernel(x, edge_index_sl, edge_index_sr,
           Wrel1_sl, brel1_sl, Wroot1_sl,
           Wrel1_sr, brel1_sr, Wroot1_sr,
           Wrel2_sl, brel2_sl, Wroot2_sl,
           Wrel2_sr, brel2_sr, Wroot2_sr,
           bn1_g, bn1_b, bn2_g, bn2_b):
    ssl = edge_index_sl[0]
    dsl = edge_index_sl[1]
    ssr = edge_index_sr[0]
    dsr = edge_index_sr[1]

    a1 = _sc_agg(x.reshape(2 * N, HALF), ssl, dsl, ssr, dsr)
    h = _tc_layer(x, *a1, Wrel1_sl, Wrel1_sr, Wroot1_sl, Wroot1_sr,
                  brel1_sl, brel1_sr, bn1_g, bn1_b, jnp.float32)

    a2 = _sc_agg(h.reshape(2 * N, HALF), ssl, dsl, ssr, dsr)
    return _tc_layer(h, *a2, Wrel2_sl, Wrel2_sr, Wroot2_sl, Wroot2_sr,
                     brel2_sl, brel2_sr, bn2_g, bn2_b, jnp.float32)
